# baseline (speedup 1.0000x reference)
import math

import jax
import jax.numpy as jnp
from jax import lax
from jax.experimental import pallas as pl
from jax.experimental.pallas import tpu as pltpu
from jax.experimental.pallas import tpu_sc as plsc

_SMOOTH = 0.1
_CONF = 1.0 - _SMOOTH

_NC, _NS = 2, 16
_NW = _NC * _NS


def _sc_gather_body(x_ref, t_ref, out_ref, t_v, row_v, rows_v, sem):
    n = t_v.shape[0]
    gpr = x_ref.shape[0] // (n * _NW)
    wid = lax.axis_index("s") * _NC + lax.axis_index("c")
    base = wid * n
    pltpu.sync_copy(t_ref.at[pl.ds(base, n)], t_v)
    for j in range(n // 16):
        tv = t_v[pl.ds(j * 16, 16)]
        s_vec = base + j * 16 + lax.iota(jnp.int32, 16)
        row_v[pl.ds(j * 16, 16)] = s_vec * gpr + (tv >> 7)
    pltpu.async_copy(x_ref.at[row_v], rows_v, sem).wait()
    pltpu.sync_copy(rows_v, out_ref.at[pl.ds(base, n)])


def _loss_body(t_ref, xtr_ref, x_ref, out_ref):
    i = pl.program_id(0)
    bs, v = x_ref.shape
    eps = _SMOOTH / (v - 1)
    k_const = (v - 2) * eps * math.log(eps) + (_CONF + eps) * math.log(_CONF + eps)

    x = x_ref[...]
    t = t_ref[0, pl.ds(i * bs, bs)]
    xtr = xtr_ref[pl.ds(i * bs, bs), :]
    lane = lax.broadcasted_iota(jnp.int32, xtr.shape, 1)
    xt = jnp.sum(jnp.where(lane == (t & 127)[:, None], xtr, 0.0), axis=1)
    rowsum = jnp.sum(x, axis=1)
    x0 = x[:, 0]
    contrib = jnp.where(t != 0, k_const - eps * (rowsum - x0) - _CONF * xt, 0.0)

    @pl.when(i == 0)
    def _():
        out_ref[0, 0] = 0.0

    out_ref[0, 0] += jnp.sum(contrib)


def kernel(x, target):
    b, s, v = x.shape
    n_tok = b * s
    tok_per_w = n_tok // _NW
    xg = x.reshape(n_tok * v // 128, 128)
    t1 = target.reshape(n_tok).astype(jnp.int32)

    sc_gather = pl.kernel(
        _sc_gather_body,
        out_type=jax.ShapeDtypeStruct((n_tok, 128), jnp.float32),
        mesh=plsc.VectorSubcoreMesh(
            core_axis_name="c", subcore_axis_name="s",
            num_cores=_NC, num_subcores=_NS,
        ),
        scratch_types=[
            pltpu.VMEM((tok_per_w,), jnp.int32),
            pltpu.VMEM((tok_per_w,), jnp.int32),
            pltpu.VMEM((tok_per_w, 128), jnp.float32),
            pltpu.SemaphoreType.DMA,
        ],
    )
    xtr = sc_gather(xg, t1)

    x2 = x.reshape(n_tok, v)
    t2 = target.reshape(1, n_tok).astype(jnp.int32)
    bs = 128
    out = pl.pallas_call(
        _loss_body,
        grid=(n_tok // bs,),
        in_specs=[
            pl.BlockSpec((1, n_tok), lambda i: (0, 0)),
            pl.BlockSpec((n_tok, 128), lambda i: (0, 0)),
            pl.BlockSpec((bs, v), lambda i: (i, 0)),
        ],
        out_specs=pl.BlockSpec(memory_space=pltpu.SMEM),
        out_shape=jax.ShapeDtypeStruct((1, 1), jnp.float32),
        compiler_params=pltpu.CompilerParams(
            dimension_semantics=("arbitrary",),
        ),
    )(t2, xtr, x2)
    return out[0, 0]


# speedup vs baseline: 1.6153x; 1.6153x over previous
import functools
import math

import jax
import jax.numpy as jnp
from jax import lax
from jax.experimental import pallas as pl
from jax.experimental.pallas import tpu as pltpu
from jax.experimental.pallas import tpu_sc as plsc

_SMOOTH = 0.1
_CONF = 1.0 - _SMOOTH

_NC, _NS = 2, 16
_NW = _NC * _NS
_SC_ROWS = 512
_ROWS_PER_W = _SC_ROWS // _NW
_CB = 6400


def _sc_body(x_ref, t_ref, out_ref, t_v, buf, res_v, sem):
    v = x_ref.shape[1]
    eps = _SMOOTH / (v - 1)
    k_const = (v - 2) * eps * math.log(eps) + (_CONF + eps) * math.log(_CONF + eps)
    nr = _ROWS_PER_W
    wid = lax.axis_index("s") * _NC + lax.axis_index("c")
    r0 = wid * nr

    pltpu.sync_copy(t_ref.at[pl.ds(r0, nr)], t_v)
    tvec = t_v[...]
    iot = lax.iota(jnp.int32, 16)

    accs = [jnp.zeros((16,), jnp.float32) for _ in range(nr)]
    x0vec = jnp.zeros((16,), jnp.float32)
    xtvec = jnp.zeros((16,), jnp.float32)
    for c in range(v // _CB):
        c0 = c * _CB
        pltpu.async_copy(
            x_ref.at[pl.ds(r0, nr), pl.ds(c0, _CB)], buf, sem
        ).wait()
        for k in range(nr):
            def step(j, acc, k=k):
                return acc + buf[k, pl.ds(j * 16, 16)]
            accs[k] = lax.fori_loop(0, _CB // 16, step, accs[k])
            tk = tvec[k]
            if c == 0:
                v0 = buf[k, pl.ds(0, 16)]
                x0vec = jnp.where(iot == k, v0[0], x0vec)
            inb = (tk >= c0) & (tk < c0 + _CB)
            rel = jnp.clip(tk - c0, 0, _CB - 1)
            base = pl.multiple_of((rel >> 4) << 4, 16)
            vv = buf[k, pl.ds(base, 16)]
            val = jnp.sum(vv * (iot == (rel & 15)).astype(jnp.float32))
            xtvec = jnp.where((iot == k) & inb, val, xtvec)

    tot = jnp.zeros((16,), jnp.float32)
    for k in range(nr):
        tot = jnp.where(iot == k, jnp.sum(accs[k]), tot)
    m = jnp.where(tvec == 0, 0.0, 1.0)
    res_v[...] = m * (k_const - eps * (tot - x0vec) - _CONF * xtvec)
    pltpu.sync_copy(res_v, out_ref.at[wid])


def _loss_body(row0, t_ref, x_ref, out_ref):
    i = pl.program_id(0)
    bs, v = x_ref.shape
    eps = _SMOOTH / (v - 1)
    k_const = (v - 2) * eps * math.log(eps) + (_CONF + eps) * math.log(_CONF + eps)

    x = x_ref[...]
    t = t_ref[0, pl.ds(row0 + i * bs, bs)]
    rowsum = jnp.sum(x, axis=1)
    x0 = x[:, 0]
    col = lax.broadcasted_iota(jnp.int32, x.shape, 1)
    xt = jnp.sum(jnp.where(col == t[:, None], x, 0.0), axis=1)
    contrib = jnp.where(t != 0, k_const - eps * (rowsum - x0) - _CONF * xt, 0.0)

    @pl.when(i == 0)
    def _():
        out_ref[0, 0] = 0.0

    out_ref[0, 0] += jnp.sum(contrib)


def _combine_body(sc_ref, tc_ref, out_ref):
    out_ref[0, 0] = tc_ref[0, 0] + jnp.sum(sc_ref[...])


def kernel(x, target):
    b, s, v = x.shape
    n_tok = b * s
    x2 = x.reshape(n_tok, v)
    t1 = target.reshape(n_tok).astype(jnp.int32)
    t2 = target.reshape(1, n_tok).astype(jnp.int32)

    sc_part = pl.kernel(
        _sc_body,
        out_type=jax.ShapeDtypeStruct((_NW, 16), jnp.float32),
        mesh=plsc.VectorSubcoreMesh(
            core_axis_name="c", subcore_axis_name="s",
            num_cores=_NC, num_subcores=_NS,
        ),
        compiler_params=pltpu.CompilerParams(needs_layout_passes=False),
        scratch_types=[
            pltpu.VMEM((_ROWS_PER_W,), jnp.int32),
            pltpu.VMEM((_ROWS_PER_W, _CB), jnp.float32),
            pltpu.VMEM((16,), jnp.float32),
            pltpu.SemaphoreType.DMA,
        ],
    )(x2, t1)

    bs = 128
    row_blk0 = _SC_ROWS // bs
    tc_part = pl.pallas_call(
        functools.partial(_loss_body, _SC_ROWS),
        grid=((n_tok - _SC_ROWS) // bs,),
        in_specs=[
            pl.BlockSpec((1, n_tok), lambda i: (0, 0)),
            pl.BlockSpec((bs, v), lambda i: (i + row_blk0, 0)),
        ],
        out_specs=pl.BlockSpec(memory_space=pltpu.SMEM),
        out_shape=jax.ShapeDtypeStruct((1, 1), jnp.float32),
        compiler_params=pltpu.CompilerParams(
            dimension_semantics=("arbitrary",),
        ),
    )(t2, x2)

    out = pl.pallas_call(
        _combine_body,
        in_specs=[
            pl.BlockSpec((_NW, 16), lambda: (0, 0)),
            pl.BlockSpec(memory_space=pltpu.SMEM),
        ],
        out_specs=pl.BlockSpec(memory_space=pltpu.SMEM),
        out_shape=jax.ShapeDtypeStruct((1, 1), jnp.float32),
    )(sc_part, tc_part)
    return out[0, 0]


# speedup vs baseline: 2.7828x; 1.7228x over previous
import functools
import math

import jax
import jax.numpy as jnp
from jax import lax
from jax.experimental import pallas as pl
from jax.experimental.pallas import tpu as pltpu
from jax.experimental.pallas import tpu_sc as plsc

_SMOOTH = 0.1
_CONF = 1.0 - _SMOOTH

_NC, _NS = 2, 16
_NW = _NC * _NS
_SC_ROWS = 256
_ROWS_PER_W = _SC_ROWS // _NW
_CB = 6400
_UNROLL = 16


def _sc_body(x_ref, t_ref, out_ref, t_v, buf0, buf1, res_v, sem0, sem1):
    v = x_ref.shape[1]
    eps = _SMOOTH / (v - 1)
    k_const = (v - 2) * eps * math.log(eps) + (_CONF + eps) * math.log(_CONF + eps)
    nr = _ROWS_PER_W
    wid = lax.axis_index("s") * _NC + lax.axis_index("c")
    r0 = wid * nr

    pltpu.sync_copy(t_ref.at[pl.ds(r0, 16)], t_v)
    tvec = t_v[...]
    iot = lax.iota(jnp.int32, 16)

    accs = [jnp.zeros((16,), jnp.float32) for _ in range(nr)]
    x0vec = jnp.zeros((16,), jnp.float32)
    xtvec = jnp.zeros((16,), jnp.float32)
    nch = v // _CB
    bufs = (buf0, buf1)
    sems = (sem0, sem1)
    handles = [None, None]
    handles[0] = pltpu.async_copy(
        x_ref.at[pl.ds(r0, nr), pl.ds(0, _CB)], buf0, sem0
    )
    for c in range(nch):
        cur = c & 1
        c0 = c * _CB
        handles[cur].wait()
        if c + 1 < nch:
            handles[1 - cur] = pltpu.async_copy(
                x_ref.at[pl.ds(r0, nr), pl.ds(c0 + _CB, _CB)],
                bufs[1 - cur], sems[1 - cur],
            )
        buf = bufs[cur]

        def step(j, carry, buf=buf):
            acc_l = list(carry)
            for k in range(nr):
                for u in range(_UNROLL):
                    acc_l[k] = acc_l[k] + buf[k, pl.ds((j * _UNROLL + u) * 16, 16)]
            return tuple(acc_l)

        accs = list(lax.fori_loop(0, _CB // (16 * _UNROLL), step, tuple(accs)))

        for k in range(nr):
            tk = tvec[k]
            if c == 0:
                v0 = buf[k, pl.ds(0, 16)]
                x0vec = jnp.where(iot == k, v0[0], x0vec)
            inb = (tk >= c0) & (tk < c0 + _CB)
            rel = jnp.clip(tk - c0, 0, _CB - 1)
            base = pl.multiple_of((rel >> 4) << 4, 16)
            vv = buf[k, pl.ds(base, 16)]
            val = jnp.sum(vv * (iot == (rel & 15)).astype(jnp.float32))
            xtvec = jnp.where((iot == k) & inb, val, xtvec)

    tot = jnp.zeros((16,), jnp.float32)
    for k in range(nr):
        tot = jnp.where(iot == k, jnp.sum(accs[k]), tot)
    m = jnp.where((tvec == 0) | (iot >= nr), 0.0, 1.0)
    res_v[...] = m * (k_const - eps * (tot - x0vec) - _CONF * xtvec)
    pltpu.sync_copy(res_v, out_ref.at[wid])


def _loss_body(row0, t_ref, x_ref, out_ref):
    i = pl.program_id(0)
    bs, v = x_ref.shape
    eps = _SMOOTH / (v - 1)
    k_const = (v - 2) * eps * math.log(eps) + (_CONF + eps) * math.log(_CONF + eps)

    x = x_ref[...]
    t = t_ref[0, pl.ds(row0 + i * bs, bs)]
    rowsum = jnp.sum(x, axis=1)
    x0 = x[:, 0]
    col = lax.broadcasted_iota(jnp.int32, x.shape, 1)
    xt = jnp.sum(jnp.where(col == t[:, None], x, 0.0), axis=1)
    contrib = jnp.where(t != 0, k_const - eps * (rowsum - x0) - _CONF * xt, 0.0)

    @pl.when(i == 0)
    def _():
        out_ref[0, 0] = 0.0

    out_ref[0, 0] += jnp.sum(contrib)


def _combine_body(sc_ref, tc_ref, out_ref):
    out_ref[0, 0] = tc_ref[0, 0] + jnp.sum(sc_ref[...])


def kernel(x, target):
    b, s, v = x.shape
    n_tok = b * s
    x2 = x.reshape(n_tok, v)
    t1 = target.reshape(n_tok).astype(jnp.int32)
    t2 = target.reshape(1, n_tok).astype(jnp.int32)

    sc_part = pl.kernel(
        _sc_body,
        out_type=jax.ShapeDtypeStruct((_NW, 16), jnp.float32),
        mesh=plsc.VectorSubcoreMesh(
            core_axis_name="c", subcore_axis_name="s",
            num_cores=_NC, num_subcores=_NS,
        ),
        compiler_params=pltpu.CompilerParams(needs_layout_passes=False),
        scratch_types=[
            pltpu.VMEM((16,), jnp.int32),
            pltpu.VMEM((_ROWS_PER_W, _CB), jnp.float32),
            pltpu.VMEM((_ROWS_PER_W, _CB), jnp.float32),
            pltpu.VMEM((16,), jnp.float32),
            pltpu.SemaphoreType.DMA,
            pltpu.SemaphoreType.DMA,
        ],
    )(x2, t1)

    bs = 128
    row_blk0 = _SC_ROWS // bs
    tc_part = pl.pallas_call(
        functools.partial(_loss_body, _SC_ROWS),
        grid=((n_tok - _SC_ROWS) // bs,),
        in_specs=[
            pl.BlockSpec((1, n_tok), lambda i: (0, 0)),
            pl.BlockSpec((bs, v), lambda i: (i + row_blk0, 0)),
        ],
        out_specs=pl.BlockSpec(memory_space=pltpu.SMEM),
        out_shape=jax.ShapeDtypeStruct((1, 1), jnp.float32),
        compiler_params=pltpu.CompilerParams(
            dimension_semantics=("arbitrary",),
        ),
    )(t2, x2)

    out = pl.pallas_call(
        _combine_body,
        in_specs=[
            pl.BlockSpec((_NW, 16), lambda: (0, 0)),
            pl.BlockSpec(memory_space=pltpu.SMEM),
        ],
        out_specs=pl.BlockSpec(memory_space=pltpu.SMEM),
        out_shape=jax.ShapeDtypeStruct((1, 1), jnp.float32),
    )(sc_part, tc_part)
    return out[0, 0]
